# baseline (speedup 1.0000x reference)
import jax
import jax.numpy as jnp
from jax.experimental import pallas as pl
from jax.experimental.pallas import tpu as pltpu

_N_CAT = 18
_COMMIT_W = 0.25
_IN = 786
_IN_PAD = 896
_D = 128
_K = 1024
_BLK = 256
_UBLK = 256

_HI = jax.lax.Precision.HIGHEST


def _dot(a, b):
    return jax.lax.dot_general(a, b, (((1,), (0,)), ((), ())),
                               precision=_HI,
                               preferred_element_type=jnp.float32)


def _silu(h):
    return h * (1.0 / (1.0 + jnp.exp(-h)))


def _rq_body(t_ref, x_ref,
             ew0, eb0, ew1, eb1, ew2, eb2, ew3, eb3,
             dw0, db0, dw1, db1, dw2, db2, dw3, db3,
             cbt0, cb0, cbt1, cb1, cbt2, cb2,
             rsum_ref, qsum_ref, en_ref, key_ref):
    i = pl.program_id(0)

    x = x_ref[...]
    h = _silu(_dot(x, ew0[...]) + eb0[...])
    h = _silu(_dot(h, ew1[...]) + eb1[...])
    h = _silu(_dot(h, ew2[...]) + eb2[...])
    z = _dot(h, ew3[...]) + eb3[...]

    rt = 1.0 / t_ref[...]
    r = z
    zhat = jnp.zeros_like(z)
    qloss = jnp.zeros((_BLK, 1), jnp.float32)
    key = jnp.zeros((_BLK, 1), jnp.int32)
    norms = []
    for cbt_ref, cb_ref in ((cbt0, cb0), (cbt1, cb1), (cbt2, cb2)):
        cbt = cbt_ref[...]
        cn = jnp.sum(cbt * cbt, axis=0, keepdims=True)
        rsq = jnp.sum(r * r, axis=1, keepdims=True)
        d = rsq - 2.0 * _dot(r, cbt) + cn
        m = jnp.min(d, axis=1, keepdims=True)
        e = jnp.exp((m - d) * rt)
        w = e / jnp.sum(e, axis=1, keepdims=True)
        emb = _dot(w, cb_ref[...])
        lane = jax.lax.broadcasted_iota(jnp.int32, d.shape, 1)
        ids = jnp.min(jnp.where(d <= m, lane, _K), axis=1, keepdims=True)
        key = key * _K + ids
        diff = emb - r
        qloss = qloss + (1.0 + _COMMIT_W) * jnp.sum(diff * diff, axis=1,
                                                    keepdims=True)
        norms.append(jnp.sqrt(jnp.sum(emb * emb, axis=1, keepdims=True)))
        r = r - emb
        zhat = zhat + emb

    g = _silu(_dot(zhat, dw0[...]) + db0[...])
    g = _silu(_dot(g, dw1[...]) + db1[...])
    g = _silu(_dot(g, dw2[...]) + db2[...])
    xh = _dot(g, dw3[...]) + db3[...]

    nrm = jnp.sqrt(jnp.sum(xh * xh, axis=1, keepdims=True))
    xh = xh / (nrm + 1e-12)
    col = jax.lax.broadcasted_iota(jnp.int32, xh.shape, 1)
    cont_mask = col < (_IN - _N_CAT)
    cat_mask = (col >= (_IN - _N_CAT)) & (col < _IN)
    xc = jnp.where(cont_mask, xh, 0.0)
    cnrm = jnp.sqrt(jnp.sum(xc * xc, axis=1, keepdims=True))
    xcn = xc / (cnrm + 1e-12)
    dm = jnp.where(cont_mask, xcn - x, 0.0)
    mse = jnp.sum(dm * dm, axis=1, keepdims=True)
    bce_el = (jnp.maximum(xh, 0.0) - xh * x
              + jnp.log(1.0 + jnp.exp(-jnp.abs(xh))))
    bce = jnp.sum(jnp.where(cat_mask, bce_el, 0.0), axis=1, keepdims=True)
    recon = mse + bce

    @pl.when(i == 0)
    def _():
        rsum_ref[...] = jnp.zeros_like(rsum_ref)
        qsum_ref[...] = jnp.zeros_like(qsum_ref)

    rsum_ref[...] += jnp.sum(recon, axis=0, keepdims=True)
    qsum_ref[...] += jnp.sum(qloss, axis=0, keepdims=True)
    en_ref[...] = jnp.concatenate(norms, axis=1)
    key_ref[...] = key


def _uniq_body(keys_ref, keysT_ref, out_ref):
    i = pl.program_id(0)
    ki = keys_ref[...]
    kj = keysT_ref[...]
    gi = i * _UBLK + jax.lax.broadcasted_iota(jnp.int32, (_UBLK, 1), 0)
    j = jax.lax.broadcasted_iota(jnp.int32, (_UBLK, kj.shape[1]), 1)
    dup = jnp.where((ki == kj) & (j > gi), 1.0, 0.0)
    hasdup = jnp.max(dup, axis=1, keepdims=True)
    cnt = jnp.sum(1.0 - hasdup, axis=0, keepdims=True)

    @pl.when(i == 0)
    def _():
        out_ref[...] = jnp.zeros_like(out_ref)

    out_ref[...] += cnt

    @pl.when(i == pl.num_programs(0) - 1)
    def _():
        out_ref[...] = out_ref[...] / jnp.float32(kj.shape[1])


def kernel(x, gumbel_t,
           enc_W0, enc_b0, enc_W1, enc_b1, enc_W2, enc_b2, enc_W3, enc_b3,
           dec_W0, dec_b0, dec_W1, dec_b1, dec_W2, dec_b2, dec_W3, dec_b3,
           cb0, cb1, cb2):
    B = x.shape[0]
    pad = _IN_PAD - _IN
    xp = jnp.pad(x, ((0, 0), (0, pad)))
    ew0 = jnp.pad(enc_W0, ((0, 0), (0, pad))).T
    ew1, ew2, ew3 = enc_W1.T, enc_W2.T, enc_W3.T
    dw0, dw1, dw2 = dec_W0.T, dec_W1.T, dec_W2.T
    dw3 = jnp.pad(dec_W3, ((0, pad), (0, 0))).T
    db3 = jnp.pad(dec_b3, (0, pad))
    row = lambda v: v.reshape(1, -1)
    t = jnp.asarray(gumbel_t, jnp.float32).reshape(1, 1)

    nb = B // _BLK
    rep = lambda shape: pl.BlockSpec(shape, lambda i: (0, 0))
    blk = lambda shape: pl.BlockSpec(shape, lambda i: (i, 0))
    in_specs = [
        rep((1, 1)),
        blk((_BLK, _IN_PAD)),
        rep((_IN_PAD, 1024)), rep((1, 1024)),
        rep((1024, 512)), rep((1, 512)),
        rep((512, 256)), rep((1, 256)),
        rep((256, _D)), rep((1, _D)),
        rep((_D, 256)), rep((1, 256)),
        rep((256, 512)), rep((1, 512)),
        rep((512, 1024)), rep((1, 1024)),
        rep((1024, _IN_PAD)), rep((1, _IN_PAD)),
        rep((_D, _K)), rep((_K, _D)),
        rep((_D, _K)), rep((_K, _D)),
        rep((_D, _K)), rep((_K, _D)),
    ]
    out_specs = [
        rep((1, 1)),
        rep((1, 1)),
        blk((_BLK, 3)),
        blk((_BLK, 1)),
    ]
    out_shape = [
        jax.ShapeDtypeStruct((1, 1), jnp.float32),
        jax.ShapeDtypeStruct((1, 1), jnp.float32),
        jax.ShapeDtypeStruct((B, 3), jnp.float32),
        jax.ShapeDtypeStruct((B, 1), jnp.int32),
    ]
    rsum, qsum, en, keys = pl.pallas_call(
        _rq_body,
        grid=(nb,),
        in_specs=in_specs,
        out_specs=out_specs,
        out_shape=out_shape,
        compiler_params=pltpu.CompilerParams(
            dimension_semantics=("arbitrary",)),
    )(t, xp,
      ew0, row(enc_b0), ew1, row(enc_b1), ew2, row(enc_b2), ew3, row(enc_b3),
      dw0, row(dec_b0), dw1, row(dec_b1), dw2, row(dec_b2), dw3, row(db3),
      cb0.T, cb0, cb1.T, cb1, cb2.T, cb2)

    keysT = keys.reshape(1, B)
    pu = pl.pallas_call(
        _uniq_body,
        grid=(B // _UBLK,),
        in_specs=[
            pl.BlockSpec((_UBLK, 1), lambda i: (i, 0)),
            pl.BlockSpec((1, B), lambda i: (0, 0)),
        ],
        out_specs=pl.BlockSpec((1, 1), lambda i: (0, 0)),
        out_shape=jax.ShapeDtypeStruct((1, 1), jnp.float32),
        compiler_params=pltpu.CompilerParams(
            dimension_semantics=("arbitrary",)),
    )(keys, keysT)

    recon_mean = rsum[0, 0] / B
    q_mean = qsum[0, 0] / B
    loss = recon_mean + q_mean
    return (loss, recon_mean, q_mean, en, pu[0, 0])


# speedup vs baseline: 2.5519x; 2.5519x over previous
import jax
import jax.numpy as jnp
from jax.experimental import pallas as pl
from jax.experimental.pallas import tpu as pltpu

_N_CAT = 18
_COMMIT_W = 0.25
_IN = 786
_IN_PAD = 896
_D = 128
_K = 1024
_BLK = 256
_UBLK = 256

_HI = jax.lax.Precision.HIGHEST


def _dot(a, b):
    return jax.lax.dot_general(a.astype(jnp.bfloat16), b.astype(jnp.bfloat16),
                               (((1,), (0,)), ((), ())),
                               preferred_element_type=jnp.float32)


def _silu(h):
    return h * (1.0 / (1.0 + jnp.exp(-h)))


def _rq_body(t_ref, x_ref,
             ew0, eb0, ew1, eb1, ew2, eb2, ew3, eb3,
             dw0, db0, dw1, db1, dw2, db2, dw3, db3,
             cbt0, cb0, cbt1, cb1, cbt2, cb2,
             rsum_ref, qsum_ref, en_ref, key_ref):
    i = pl.program_id(0)

    x = x_ref[...]
    h = _silu(_dot(x, ew0[...]) + eb0[...])
    h = _silu(_dot(h, ew1[...]) + eb1[...])
    h = _silu(_dot(h, ew2[...]) + eb2[...])
    z = _dot(h, ew3[...]) + eb3[...]

    rt = 1.0 / t_ref[...]
    r = z
    zhat = jnp.zeros_like(z)
    qloss = jnp.zeros((_BLK, 1), jnp.float32)
    key = jnp.zeros((_BLK, 1), jnp.int32)
    norms = []
    for cbt_ref, cb_ref in ((cbt0, cb0), (cbt1, cb1), (cbt2, cb2)):
        cbt = cbt_ref[...]
        cn = jnp.sum(cbt * cbt, axis=0, keepdims=True)
        rsq = jnp.sum(r * r, axis=1, keepdims=True)
        d = rsq - 2.0 * _dot(r, cbt) + cn
        m = jnp.min(d, axis=1, keepdims=True)
        e = jnp.exp((m - d) * rt)
        w = e / jnp.sum(e, axis=1, keepdims=True)
        emb = _dot(w, cb_ref[...])
        lane = jax.lax.broadcasted_iota(jnp.int32, d.shape, 1)
        ids = jnp.min(jnp.where(d <= m, lane, _K), axis=1, keepdims=True)
        key = key * _K + ids
        diff = emb - r
        qloss = qloss + (1.0 + _COMMIT_W) * jnp.sum(diff * diff, axis=1,
                                                    keepdims=True)
        norms.append(jnp.sqrt(jnp.sum(emb * emb, axis=1, keepdims=True)))
        r = r - emb
        zhat = zhat + emb

    g = _silu(_dot(zhat, dw0[...]) + db0[...])
    g = _silu(_dot(g, dw1[...]) + db1[...])
    g = _silu(_dot(g, dw2[...]) + db2[...])
    xh = _dot(g, dw3[...]) + db3[...]

    nrm = jnp.sqrt(jnp.sum(xh * xh, axis=1, keepdims=True))
    xh = xh / (nrm + 1e-12)
    col = jax.lax.broadcasted_iota(jnp.int32, xh.shape, 1)
    cont_mask = col < (_IN - _N_CAT)
    cat_mask = (col >= (_IN - _N_CAT)) & (col < _IN)
    xc = jnp.where(cont_mask, xh, 0.0)
    cnrm = jnp.sqrt(jnp.sum(xc * xc, axis=1, keepdims=True))
    xcn = xc / (cnrm + 1e-12)
    dm = jnp.where(cont_mask, xcn - x, 0.0)
    mse = jnp.sum(dm * dm, axis=1, keepdims=True)
    bce_el = (jnp.maximum(xh, 0.0) - xh * x
              + jnp.log(1.0 + jnp.exp(-jnp.abs(xh))))
    bce = jnp.sum(jnp.where(cat_mask, bce_el, 0.0), axis=1, keepdims=True)
    recon = mse + bce

    @pl.when(i == 0)
    def _():
        rsum_ref[...] = jnp.zeros_like(rsum_ref)
        qsum_ref[...] = jnp.zeros_like(qsum_ref)

    rsum_ref[...] += jnp.sum(recon, axis=0, keepdims=True)
    qsum_ref[...] += jnp.sum(qloss, axis=0, keepdims=True)
    en_ref[...] = jnp.concatenate(norms, axis=1)
    key_ref[...] = key


def _uniq_body(keys_ref, keysT_ref, out_ref):
    i = pl.program_id(0)
    ki = keys_ref[...]
    kj = keysT_ref[...]
    gi = i * _UBLK + jax.lax.broadcasted_iota(jnp.int32, (_UBLK, 1), 0)
    j = jax.lax.broadcasted_iota(jnp.int32, (_UBLK, kj.shape[1]), 1)
    dup = jnp.where((ki == kj) & (j > gi), 1.0, 0.0)
    hasdup = jnp.max(dup, axis=1, keepdims=True)
    cnt = jnp.sum(1.0 - hasdup, axis=0, keepdims=True)

    @pl.when(i == 0)
    def _():
        out_ref[...] = jnp.zeros_like(out_ref)

    out_ref[...] += cnt

    @pl.when(i == pl.num_programs(0) - 1)
    def _():
        out_ref[...] = out_ref[...] / jnp.float32(kj.shape[1])


def kernel(x, gumbel_t,
           enc_W0, enc_b0, enc_W1, enc_b1, enc_W2, enc_b2, enc_W3, enc_b3,
           dec_W0, dec_b0, dec_W1, dec_b1, dec_W2, dec_b2, dec_W3, dec_b3,
           cb0, cb1, cb2):
    B = x.shape[0]
    pad = _IN_PAD - _IN
    xp = jnp.pad(x, ((0, 0), (0, pad)))
    ew0 = jnp.pad(enc_W0, ((0, 0), (0, pad))).T
    ew1, ew2, ew3 = enc_W1.T, enc_W2.T, enc_W3.T
    dw0, dw1, dw2 = dec_W0.T, dec_W1.T, dec_W2.T
    dw3 = jnp.pad(dec_W3, ((0, pad), (0, 0))).T
    db3 = jnp.pad(dec_b3, (0, pad))
    row = lambda v: v.reshape(1, -1)
    t = jnp.asarray(gumbel_t, jnp.float32).reshape(1, 1)

    nb = B // _BLK
    rep = lambda shape: pl.BlockSpec(shape, lambda i: (0, 0))
    blk = lambda shape: pl.BlockSpec(shape, lambda i: (i, 0))
    in_specs = [
        rep((1, 1)),
        blk((_BLK, _IN_PAD)),
        rep((_IN_PAD, 1024)), rep((1, 1024)),
        rep((1024, 512)), rep((1, 512)),
        rep((512, 256)), rep((1, 256)),
        rep((256, _D)), rep((1, _D)),
        rep((_D, 256)), rep((1, 256)),
        rep((256, 512)), rep((1, 512)),
        rep((512, 1024)), rep((1, 1024)),
        rep((1024, _IN_PAD)), rep((1, _IN_PAD)),
        rep((_D, _K)), rep((_K, _D)),
        rep((_D, _K)), rep((_K, _D)),
        rep((_D, _K)), rep((_K, _D)),
    ]
    out_specs = [
        rep((1, 1)),
        rep((1, 1)),
        blk((_BLK, 3)),
        blk((_BLK, 1)),
    ]
    out_shape = [
        jax.ShapeDtypeStruct((1, 1), jnp.float32),
        jax.ShapeDtypeStruct((1, 1), jnp.float32),
        jax.ShapeDtypeStruct((B, 3), jnp.float32),
        jax.ShapeDtypeStruct((B, 1), jnp.int32),
    ]
    rsum, qsum, en, keys = pl.pallas_call(
        _rq_body,
        grid=(nb,),
        in_specs=in_specs,
        out_specs=out_specs,
        out_shape=out_shape,
        compiler_params=pltpu.CompilerParams(
            dimension_semantics=("arbitrary",)),
    )(t, xp,
      ew0, row(enc_b0), ew1, row(enc_b1), ew2, row(enc_b2), ew3, row(enc_b3),
      dw0, row(dec_b0), dw1, row(dec_b1), dw2, row(dec_b2), dw3, row(db3),
      cb0.T, cb0, cb1.T, cb1, cb2.T, cb2)

    keysT = keys.reshape(1, B)
    pu = pl.pallas_call(
        _uniq_body,
        grid=(B // _UBLK,),
        in_specs=[
            pl.BlockSpec((_UBLK, 1), lambda i: (i, 0)),
            pl.BlockSpec((1, B), lambda i: (0, 0)),
        ],
        out_specs=pl.BlockSpec((1, 1), lambda i: (0, 0)),
        out_shape=jax.ShapeDtypeStruct((1, 1), jnp.float32),
        compiler_params=pltpu.CompilerParams(
            dimension_semantics=("arbitrary",)),
    )(keys, keysT)

    recon_mean = rsum[0, 0] / B
    q_mean = qsum[0, 0] / B
    loss = recon_mean + q_mean
    return (loss, recon_mean, q_mean, en, pu[0, 0])


# speedup vs baseline: 2.5690x; 1.0067x over previous
import jax
import jax.numpy as jnp
from jax.experimental import pallas as pl
from jax.experimental.pallas import tpu as pltpu

_N_CAT = 18
_COMMIT_W = 0.25
_IN = 786
_IN_PAD = 896
_D = 128
_K = 1024
_BLK = 256
_UBLK = 256

_HI = jax.lax.Precision.HIGHEST


def _dot(a, b):
    return jax.lax.dot_general(a.astype(jnp.bfloat16), b.astype(jnp.bfloat16),
                               (((1,), (0,)), ((), ())),
                               preferred_element_type=jnp.float32)


def _silu(h):
    return h * (0.5 * jnp.tanh(0.5 * h) + 0.5)


def _rq_body(t_ref, x_ref,
             ew0, eb0, ew1, eb1, ew2, eb2, ew3, eb3,
             dw0, db0, dw1, db1, dw2, db2, dw3, db3,
             cbt0, cb0, cbt1, cb1, cbt2, cb2,
             rsum_ref, qsum_ref, en_ref, key_ref):
    i = pl.program_id(0)

    x = x_ref[...]
    h = _silu(_dot(x, ew0[...]) + eb0[...])
    h = _silu(_dot(h, ew1[...]) + eb1[...])
    h = _silu(_dot(h, ew2[...]) + eb2[...])
    z = _dot(h, ew3[...]) + eb3[...]

    rt = 1.0 / t_ref[...]
    r = z
    zhat = jnp.zeros_like(z)
    qloss = jnp.zeros((_BLK, 1), jnp.float32)
    key = jnp.zeros((_BLK, 1), jnp.int32)
    norms = []
    for cbt_ref, cb_ref in ((cbt0, cb0), (cbt1, cb1), (cbt2, cb2)):
        cbt = cbt_ref[...]
        cn = jnp.sum(cbt * cbt, axis=0, keepdims=True)
        rsq = jnp.sum(r * r, axis=1, keepdims=True)
        d = rsq - 2.0 * _dot(r, cbt) + cn
        m = jnp.min(d, axis=1, keepdims=True)
        e = jnp.exp((m - d) * rt)
        w = e * (1.0 / jnp.sum(e, axis=1, keepdims=True))
        emb = _dot(w, cb_ref[...])
        lane = jax.lax.broadcasted_iota(jnp.int32, d.shape, 1)
        ids = jnp.min(jnp.where(d <= m, lane, _K), axis=1, keepdims=True)
        key = key * _K + ids
        diff = emb - r
        qloss = qloss + (1.0 + _COMMIT_W) * jnp.sum(diff * diff, axis=1,
                                                    keepdims=True)
        norms.append(jnp.sqrt(jnp.sum(emb * emb, axis=1, keepdims=True)))
        r = r - emb
        zhat = zhat + emb

    g = _silu(_dot(zhat, dw0[...]) + db0[...])
    g = _silu(_dot(g, dw1[...]) + db1[...])
    g = _silu(_dot(g, dw2[...]) + db2[...])
    xh = _dot(g, dw3[...]) + db3[...]

    nrm = jnp.sqrt(jnp.sum(xh * xh, axis=1, keepdims=True))
    xh = xh * (1.0 / (nrm + 1e-12))
    nc = _IN - _N_CAT
    xc = xh[:, :nc]
    cnrm = jnp.sqrt(jnp.sum(xc * xc, axis=1, keepdims=True))
    dm = xc * (1.0 / (cnrm + 1e-12)) - x[:, :nc]
    mse = jnp.sum(dm * dm, axis=1, keepdims=True)
    lg = xh[:, nc:]
    tg = x[:, nc:]
    bce_el = (jnp.maximum(lg, 0.0) - lg * tg
              + jnp.log(1.0 + jnp.exp(-jnp.abs(lg))))
    ccol = jax.lax.broadcasted_iota(jnp.int32, bce_el.shape, 1)
    bce = jnp.sum(jnp.where(ccol < _N_CAT, bce_el, 0.0), axis=1,
                  keepdims=True)
    recon = mse + bce

    @pl.when(i == 0)
    def _():
        rsum_ref[...] = jnp.zeros_like(rsum_ref)
        qsum_ref[...] = jnp.zeros_like(qsum_ref)

    rsum_ref[...] += jnp.sum(recon, axis=0, keepdims=True)
    qsum_ref[...] += jnp.sum(qloss, axis=0, keepdims=True)
    en_ref[...] = jnp.concatenate(norms, axis=1)
    key_ref[...] = key


def _uniq_body(keys_ref, keysT_ref, out_ref):
    i = pl.program_id(0)
    ki = keys_ref[...]
    kj = keysT_ref[...]
    gi = i * _UBLK + jax.lax.broadcasted_iota(jnp.int32, (_UBLK, 1), 0)
    j = jax.lax.broadcasted_iota(jnp.int32, (_UBLK, kj.shape[1]), 1)
    dup = jnp.where((ki == kj) & (j > gi), 1.0, 0.0)
    hasdup = jnp.max(dup, axis=1, keepdims=True)
    cnt = jnp.sum(1.0 - hasdup, axis=0, keepdims=True)

    @pl.when(i == 0)
    def _():
        out_ref[...] = jnp.zeros_like(out_ref)

    out_ref[...] += cnt

    @pl.when(i == pl.num_programs(0) - 1)
    def _():
        out_ref[...] = out_ref[...] / jnp.float32(kj.shape[1])


def kernel(x, gumbel_t,
           enc_W0, enc_b0, enc_W1, enc_b1, enc_W2, enc_b2, enc_W3, enc_b3,
           dec_W0, dec_b0, dec_W1, dec_b1, dec_W2, dec_b2, dec_W3, dec_b3,
           cb0, cb1, cb2):
    B = x.shape[0]
    pad = _IN_PAD - _IN
    xp = jnp.pad(x, ((0, 0), (0, pad)))
    ew0 = jnp.pad(enc_W0, ((0, 0), (0, pad))).T
    ew1, ew2, ew3 = enc_W1.T, enc_W2.T, enc_W3.T
    dw0, dw1, dw2 = dec_W0.T, dec_W1.T, dec_W2.T
    dw3 = jnp.pad(dec_W3, ((0, pad), (0, 0))).T
    db3 = jnp.pad(dec_b3, (0, pad))
    row = lambda v: v.reshape(1, -1)
    t = jnp.asarray(gumbel_t, jnp.float32).reshape(1, 1)

    nb = B // _BLK
    rep = lambda shape: pl.BlockSpec(shape, lambda i: (0, 0))
    blk = lambda shape: pl.BlockSpec(shape, lambda i: (i, 0))
    in_specs = [
        rep((1, 1)),
        blk((_BLK, _IN_PAD)),
        rep((_IN_PAD, 1024)), rep((1, 1024)),
        rep((1024, 512)), rep((1, 512)),
        rep((512, 256)), rep((1, 256)),
        rep((256, _D)), rep((1, _D)),
        rep((_D, 256)), rep((1, 256)),
        rep((256, 512)), rep((1, 512)),
        rep((512, 1024)), rep((1, 1024)),
        rep((1024, _IN_PAD)), rep((1, _IN_PAD)),
        rep((_D, _K)), rep((_K, _D)),
        rep((_D, _K)), rep((_K, _D)),
        rep((_D, _K)), rep((_K, _D)),
    ]
    out_specs = [
        rep((1, 1)),
        rep((1, 1)),
        blk((_BLK, 3)),
        blk((_BLK, 1)),
    ]
    out_shape = [
        jax.ShapeDtypeStruct((1, 1), jnp.float32),
        jax.ShapeDtypeStruct((1, 1), jnp.float32),
        jax.ShapeDtypeStruct((B, 3), jnp.float32),
        jax.ShapeDtypeStruct((B, 1), jnp.int32),
    ]
    rsum, qsum, en, keys = pl.pallas_call(
        _rq_body,
        grid=(nb,),
        in_specs=in_specs,
        out_specs=out_specs,
        out_shape=out_shape,
        compiler_params=pltpu.CompilerParams(
            dimension_semantics=("arbitrary",)),
    )(t, xp,
      ew0, row(enc_b0), ew1, row(enc_b1), ew2, row(enc_b2), ew3, row(enc_b3),
      dw0, row(dec_b0), dw1, row(dec_b1), dw2, row(dec_b2), dw3, row(db3),
      cb0.T, cb0, cb1.T, cb1, cb2.T, cb2)

    keysT = keys.reshape(1, B)
    pu = pl.pallas_call(
        _uniq_body,
        grid=(B // _UBLK,),
        in_specs=[
            pl.BlockSpec((_UBLK, 1), lambda i: (i, 0)),
            pl.BlockSpec((1, B), lambda i: (0, 0)),
        ],
        out_specs=pl.BlockSpec((1, 1), lambda i: (0, 0)),
        out_shape=jax.ShapeDtypeStruct((1, 1), jnp.float32),
        compiler_params=pltpu.CompilerParams(
            dimension_semantics=("arbitrary",)),
    )(keys, keysT)

    recon_mean = rsum[0, 0] / B
    q_mean = qsum[0, 0] / B
    loss = recon_mean + q_mean
    return (loss, recon_mean, q_mean, en, pu[0, 0])


# speedup vs baseline: 3.0171x; 1.1744x over previous
import jax
import jax.numpy as jnp
from jax.experimental import pallas as pl
from jax.experimental.pallas import tpu as pltpu

_N_CAT = 18
_COMMIT_W = 0.25
_IN = 786
_D = 128
_K = 1024
_BLK = 256
_UBLK = 256


def _dot(a, b):
    return jax.lax.dot_general(a.astype(jnp.bfloat16), b.astype(jnp.bfloat16),
                               (((1,), (0,)), ((), ())),
                               preferred_element_type=jnp.float32)


def _dt(a, b):
    return jax.lax.dot_general(a.astype(jnp.bfloat16), b.astype(jnp.bfloat16),
                               (((1,), (1,)), ((), ())),
                               preferred_element_type=jnp.float32)


def _silu(h):
    return h * (0.5 * jnp.tanh(0.5 * h) + 0.5)


def _rq_body(t_ref, x_ref,
             ew0, eb0, ew1, eb1, ew2, eb2, ew3, eb3,
             dw0, db0, dw1, db1, dw2, db2, dw3, db3,
             cbt0, cb0, cbt1, cb1, cbt2, cb2,
             rsum_ref, qsum_ref, en_ref, key_ref):
    i = pl.program_id(0)

    x = x_ref[...]
    h = _silu(_dt(x, ew0[...]) + eb0[...])
    h = _silu(_dt(h, ew1[...]) + eb1[...])
    h = _silu(_dt(h, ew2[...]) + eb2[...])
    z = _dt(h, ew3[...]) + eb3[...]

    rt = 1.0 / t_ref[...]
    r = z
    zhat = jnp.zeros_like(z)
    qloss = jnp.zeros((_BLK, 1), jnp.float32)
    key = jnp.zeros((_BLK, 1), jnp.int32)
    norms = []
    for cbt_ref, cb_ref in ((cbt0, cb0), (cbt1, cb1), (cbt2, cb2)):
        cbt = cbt_ref[...]
        cn = jnp.sum(cbt * cbt, axis=0, keepdims=True)
        rsq = jnp.sum(r * r, axis=1, keepdims=True)
        d = rsq - 2.0 * _dot(r, cbt) + cn
        m = jnp.min(d, axis=1, keepdims=True)
        e = jnp.exp((m - d) * rt)
        w = e * (1.0 / jnp.sum(e, axis=1, keepdims=True))
        emb = _dot(w, cb_ref[...])
        lane = jax.lax.broadcasted_iota(jnp.int32, d.shape, 1)
        ids = jnp.min(jnp.where(d <= m, lane, _K), axis=1, keepdims=True)
        key = key * _K + ids
        diff = emb - r
        qloss = qloss + (1.0 + _COMMIT_W) * jnp.sum(diff * diff, axis=1,
                                                    keepdims=True)
        norms.append(jnp.sqrt(jnp.sum(emb * emb, axis=1, keepdims=True)))
        r = r - emb
        zhat = zhat + emb

    g = _silu(_dt(zhat, dw0[...]) + db0[...])
    g = _silu(_dt(g, dw1[...]) + db1[...])
    g = _silu(_dt(g, dw2[...]) + db2[...])
    xh = _dt(g, dw3[...]) + db3[...]

    nrm = jnp.sqrt(jnp.sum(xh * xh, axis=1, keepdims=True))
    xh = xh * (1.0 / (nrm + 1e-12))
    nc = _IN - _N_CAT
    xc = xh[:, :nc]
    cnrm = jnp.sqrt(jnp.sum(xc * xc, axis=1, keepdims=True))
    dm = xc * (1.0 / (cnrm + 1e-12)) - x[:, :nc]
    mse = jnp.sum(dm * dm, axis=1, keepdims=True)
    lg = xh[:, nc:]
    tg = x[:, nc:]
    bce_el = (jnp.maximum(lg, 0.0) - lg * tg
              + jnp.log(1.0 + jnp.exp(-jnp.abs(lg))))
    bce = jnp.sum(bce_el, axis=1, keepdims=True)
    recon = mse + bce

    @pl.when(i == 0)
    def _():
        rsum_ref[...] = jnp.zeros_like(rsum_ref)
        qsum_ref[...] = jnp.zeros_like(qsum_ref)

    rsum_ref[...] += jnp.sum(recon, axis=0, keepdims=True)
    qsum_ref[...] += jnp.sum(qloss, axis=0, keepdims=True)
    en_ref[...] = jnp.concatenate(norms, axis=1)
    key_ref[...] = key


def _uniq_body(keys_ref, keysT_ref, out_ref):
    i = pl.program_id(0)
    ki = keys_ref[...]
    kj = keysT_ref[...]
    gi = i * _UBLK + jax.lax.broadcasted_iota(jnp.int32, (_UBLK, 1), 0)
    j = jax.lax.broadcasted_iota(jnp.int32, (_UBLK, kj.shape[1]), 1)
    dup = jnp.where((ki == kj) & (j > gi), 1.0, 0.0)
    hasdup = jnp.max(dup, axis=1, keepdims=True)
    cnt = jnp.sum(1.0 - hasdup, axis=0, keepdims=True)

    @pl.when(i == 0)
    def _():
        out_ref[...] = jnp.zeros_like(out_ref)

    out_ref[...] += cnt

    @pl.when(i == pl.num_programs(0) - 1)
    def _():
        out_ref[...] = out_ref[...] / jnp.float32(kj.shape[1])


def kernel(x, gumbel_t,
           enc_W0, enc_b0, enc_W1, enc_b1, enc_W2, enc_b2, enc_W3, enc_b3,
           dec_W0, dec_b0, dec_W1, dec_b1, dec_W2, dec_b2, dec_W3, dec_b3,
           cb0, cb1, cb2):
    B = x.shape[0]
    row = lambda v: v.reshape(1, -1)
    t = jnp.asarray(gumbel_t, jnp.float32).reshape(1, 1)

    nb = B // _BLK
    rep = lambda shape: pl.BlockSpec(shape, lambda i: (0, 0))
    blk = lambda shape: pl.BlockSpec(shape, lambda i: (i, 0))
    in_specs = [
        rep((1, 1)),
        blk((_BLK, _IN)),
        rep((1024, _IN)), rep((1, 1024)),
        rep((512, 1024)), rep((1, 512)),
        rep((256, 512)), rep((1, 256)),
        rep((_D, 256)), rep((1, _D)),
        rep((256, _D)), rep((1, 256)),
        rep((512, 256)), rep((1, 512)),
        rep((1024, 512)), rep((1, 1024)),
        rep((_IN, 1024)), rep((1, _IN)),
        rep((_D, _K)), rep((_K, _D)),
        rep((_D, _K)), rep((_K, _D)),
        rep((_D, _K)), rep((_K, _D)),
    ]
    out_specs = [
        rep((1, 1)),
        rep((1, 1)),
        blk((_BLK, 3)),
        blk((_BLK, 1)),
    ]
    out_shape = [
        jax.ShapeDtypeStruct((1, 1), jnp.float32),
        jax.ShapeDtypeStruct((1, 1), jnp.float32),
        jax.ShapeDtypeStruct((B, 3), jnp.float32),
        jax.ShapeDtypeStruct((B, 1), jnp.int32),
    ]
    rsum, qsum, en, keys = pl.pallas_call(
        _rq_body,
        grid=(nb,),
        in_specs=in_specs,
        out_specs=out_specs,
        out_shape=out_shape,
        compiler_params=pltpu.CompilerParams(
            dimension_semantics=("arbitrary",)),
    )(t, x,
      enc_W0, row(enc_b0), enc_W1, row(enc_b1), enc_W2, row(enc_b2),
      enc_W3, row(enc_b3),
      dec_W0, row(dec_b0), dec_W1, row(dec_b1), dec_W2, row(dec_b2),
      dec_W3, row(dec_b3),
      cb0.T, cb0, cb1.T, cb1, cb2.T, cb2)

    keysT = keys.reshape(1, B)
    pu = pl.pallas_call(
        _uniq_body,
        grid=(B // _UBLK,),
        in_specs=[
            pl.BlockSpec((_UBLK, 1), lambda i: (i, 0)),
            pl.BlockSpec((1, B), lambda i: (0, 0)),
        ],
        out_specs=pl.BlockSpec((1, 1), lambda i: (0, 0)),
        out_shape=jax.ShapeDtypeStruct((1, 1), jnp.float32),
        compiler_params=pltpu.CompilerParams(
            dimension_semantics=("arbitrary",)),
    )(keys, keysT)

    recon_mean = rsum[0, 0] / B
    q_mean = qsum[0, 0] / B
    loss = recon_mean + q_mean
    return (loss, recon_mean, q_mean, en, pu[0, 0])


# speedup vs baseline: 3.4152x; 1.1319x over previous
import jax
import jax.numpy as jnp
from jax.experimental import pallas as pl
from jax.experimental.pallas import tpu as pltpu

_N_CAT = 18
_COMMIT_W = 0.25
_IN = 786
_D = 128
_K = 1024
_BLK = 512
_UBLK = 256
_LOG2E = 1.4426950408889634


def _dot(a, b):
    return jax.lax.dot_general(a.astype(jnp.bfloat16), b.astype(jnp.bfloat16),
                               (((1,), (0,)), ((), ())),
                               preferred_element_type=jnp.float32)


def _dt(a, b):
    return jax.lax.dot_general(a.astype(jnp.bfloat16), b.astype(jnp.bfloat16),
                               (((1,), (1,)), ((), ())),
                               preferred_element_type=jnp.float32)


def _silu(h):
    u = 0.5 * h
    return u * (jnp.tanh(u) + 1.0)


def _rq_body(t_ref, x_ref,
             ew0, eb0, ew1, eb1, ew2, eb2, ew3, eb3,
             dw0, db0, dw1, db1, dw2, db2, dw3, db3,
             cbt0, cb0, cbt1, cb1, cbt2, cb2,
             rsum_ref, qsum_ref, en_ref, key_ref):
    i = pl.program_id(0)

    x = x_ref[...]
    h = _silu(_dt(x, ew0[...]) + eb0[...])
    h = _silu(_dt(h, ew1[...]) + eb1[...])
    h = _silu(_dt(h, ew2[...]) + eb2[...])
    z = _dt(h, ew3[...]) + eb3[...]

    rt2 = _LOG2E / t_ref[...]
    r = z
    zhat = jnp.zeros_like(z)
    qloss = jnp.zeros((_BLK, 1), jnp.float32)
    key = jnp.zeros((_BLK, 1), jnp.int32)
    norms = []
    for cbt_ref, cb_ref in ((cbt0, cb0), (cbt1, cb1), (cbt2, cb2)):
        cbt = cbt_ref[...]
        cn = jnp.sum(cbt * cbt, axis=0, keepdims=True)
        s = cn - 2.0 * _dot(r, cbt)
        m = jnp.min(s, axis=1, keepdims=True)
        e = jnp.exp2((m - s) * rt2)
        emb = _dot(e, cb_ref[...]) * (1.0 / jnp.sum(e, axis=1,
                                                    keepdims=True))
        lane = jax.lax.broadcasted_iota(jnp.int32, s.shape, 1)
        ids = jnp.min(jnp.where(s <= m, lane, _K), axis=1, keepdims=True)
        key = key * _K + ids
        diff = emb - r
        qloss = qloss + (1.0 + _COMMIT_W) * jnp.sum(diff * diff, axis=1,
                                                    keepdims=True)
        norms.append(jnp.sqrt(jnp.sum(emb * emb, axis=1, keepdims=True)))
        r = r - emb
        zhat = zhat + emb

    g = _silu(_dt(zhat, dw0[...]) + db0[...])
    g = _silu(_dt(g, dw1[...]) + db1[...])
    g = _silu(_dt(g, dw2[...]) + db2[...])
    xh = _dt(g, dw3[...]) + db3[...]

    nrm = jnp.sqrt(jnp.sum(xh * xh, axis=1, keepdims=True))
    xh = xh * (1.0 / (nrm + 1e-12))
    nc = _IN - _N_CAT
    xc = xh[:, :nc]
    cnrm = jnp.sqrt(jnp.sum(xc * xc, axis=1, keepdims=True))
    dm = xc * (1.0 / (cnrm + 1e-12)) - x[:, :nc]
    mse = jnp.sum(dm * dm, axis=1, keepdims=True)
    lg = xh[:, nc:]
    tg = x[:, nc:]
    bce_el = (jnp.maximum(lg, 0.0) - lg * tg
              + jnp.log(1.0 + jnp.exp(-jnp.abs(lg))))
    bce = jnp.sum(bce_el, axis=1, keepdims=True)
    recon = mse + bce

    @pl.when(i == 0)
    def _():
        rsum_ref[...] = jnp.zeros_like(rsum_ref)
        qsum_ref[...] = jnp.zeros_like(qsum_ref)

    rsum_ref[...] += jnp.sum(recon, axis=0, keepdims=True)
    qsum_ref[...] += jnp.sum(qloss, axis=0, keepdims=True)
    en_ref[...] = jnp.concatenate(norms, axis=1)
    key_ref[...] = key


def _uniq_body(keys_ref, keysT_ref, out_ref):
    i = pl.program_id(0)
    ki = keys_ref[...]
    kj = keysT_ref[...]
    gi = i * _UBLK + jax.lax.broadcasted_iota(jnp.int32, (_UBLK, 1), 0)
    j = jax.lax.broadcasted_iota(jnp.int32, (_UBLK, kj.shape[1]), 1)
    dup = jnp.where((ki == kj) & (j > gi), 1.0, 0.0)
    hasdup = jnp.max(dup, axis=1, keepdims=True)
    cnt = jnp.sum(1.0 - hasdup, axis=0, keepdims=True)

    @pl.when(i == 0)
    def _():
        out_ref[...] = jnp.zeros_like(out_ref)

    out_ref[...] += cnt

    @pl.when(i == pl.num_programs(0) - 1)
    def _():
        out_ref[...] = out_ref[...] / jnp.float32(kj.shape[1])


def kernel(x, gumbel_t,
           enc_W0, enc_b0, enc_W1, enc_b1, enc_W2, enc_b2, enc_W3, enc_b3,
           dec_W0, dec_b0, dec_W1, dec_b1, dec_W2, dec_b2, dec_W3, dec_b3,
           cb0, cb1, cb2):
    B = x.shape[0]
    row = lambda v: v.reshape(1, -1)
    bf = lambda v: v.astype(jnp.bfloat16)
    t = jnp.asarray(gumbel_t, jnp.float32).reshape(1, 1)

    nb = B // _BLK
    rep = lambda shape: pl.BlockSpec(shape, lambda i: (0, 0))
    blk = lambda shape: pl.BlockSpec(shape, lambda i: (i, 0))
    in_specs = [
        rep((1, 1)),
        blk((_BLK, _IN)),
        rep((1024, _IN)), rep((1, 1024)),
        rep((512, 1024)), rep((1, 512)),
        rep((256, 512)), rep((1, 256)),
        rep((_D, 256)), rep((1, _D)),
        rep((256, _D)), rep((1, 256)),
        rep((512, 256)), rep((1, 512)),
        rep((1024, 512)), rep((1, 1024)),
        rep((_IN, 1024)), rep((1, _IN)),
        rep((_D, _K)), rep((_K, _D)),
        rep((_D, _K)), rep((_K, _D)),
        rep((_D, _K)), rep((_K, _D)),
    ]
    out_specs = [
        rep((1, 1)),
        rep((1, 1)),
        blk((_BLK, 3)),
        blk((_BLK, 1)),
    ]
    out_shape = [
        jax.ShapeDtypeStruct((1, 1), jnp.float32),
        jax.ShapeDtypeStruct((1, 1), jnp.float32),
        jax.ShapeDtypeStruct((B, 3), jnp.float32),
        jax.ShapeDtypeStruct((B, 1), jnp.int32),
    ]
    rsum, qsum, en, keys = pl.pallas_call(
        _rq_body,
        grid=(nb,),
        in_specs=in_specs,
        out_specs=out_specs,
        out_shape=out_shape,
        compiler_params=pltpu.CompilerParams(
            dimension_semantics=("arbitrary",)),
    )(t, x,
      bf(enc_W0), row(enc_b0), bf(enc_W1), row(enc_b1),
      bf(enc_W2), row(enc_b2), bf(enc_W3), row(enc_b3),
      bf(dec_W0), row(dec_b0), bf(dec_W1), row(dec_b1),
      bf(dec_W2), row(dec_b2), bf(dec_W3), row(dec_b3),
      cb0.T, bf(cb0), cb1.T, bf(cb1), cb2.T, bf(cb2))

    keysT = keys.reshape(1, B)
    pu = pl.pallas_call(
        _uniq_body,
        grid=(B // _UBLK,),
        in_specs=[
            pl.BlockSpec((_UBLK, 1), lambda i: (i, 0)),
            pl.BlockSpec((1, B), lambda i: (0, 0)),
        ],
        out_specs=pl.BlockSpec((1, 1), lambda i: (0, 0)),
        out_shape=jax.ShapeDtypeStruct((1, 1), jnp.float32),
        compiler_params=pltpu.CompilerParams(
            dimension_semantics=("arbitrary",)),
    )(keys, keysT)

    recon_mean = rsum[0, 0] / B
    q_mean = qsum[0, 0] / B
    loss = recon_mean + q_mean
    return (loss, recon_mean, q_mean, en, pu[0, 0])


# speedup vs baseline: 3.5321x; 1.0342x over previous
import jax
import jax.numpy as jnp
from jax.experimental import pallas as pl
from jax.experimental.pallas import tpu as pltpu

_N_CAT = 18
_COMMIT_W = 0.25
_IN = 786
_D = 128
_K = 1024
_BLK = 512
_UBLK = 512
_LOG2E = 1.4426950408889634


def _dot(a, b):
    return jax.lax.dot_general(a.astype(jnp.bfloat16), b.astype(jnp.bfloat16),
                               (((1,), (0,)), ((), ())),
                               preferred_element_type=jnp.float32)


def _dt(a, b):
    return jax.lax.dot_general(a.astype(jnp.bfloat16), b.astype(jnp.bfloat16),
                               (((1,), (1,)), ((), ())),
                               preferred_element_type=jnp.float32)


def _silu(h):
    u = 0.5 * h
    return u * (jnp.tanh(u) + 1.0)


def _rq_body(t_ref, x_ref,
             ew0, eb0, ew1, eb1, ew2, eb2, ew3, eb3,
             dw0, db0, dw1, db1, dw2, db2, dw3, db3,
             cbt0, cb0, cbt1, cb1, cbt2, cb2,
             rsum_ref, qsum_ref, en_ref, key_ref):
    i = pl.program_id(0)

    x = x_ref[...]
    h = _silu(_dt(x, ew0[...]) + eb0[...])
    h = _silu(_dt(h, ew1[...]) + eb1[...])
    h = _silu(_dt(h, ew2[...]) + eb2[...])
    z = _dt(h, ew3[...]) + eb3[...]

    rt2 = -_LOG2E / t_ref[...]
    r = z
    zhat = jnp.zeros_like(z)
    qloss = jnp.zeros((_BLK, 1), jnp.float32)
    key = jnp.zeros((_BLK, 1), jnp.int32)
    norms = []
    for cbt_ref, cb_ref in ((cbt0, cb0), (cbt1, cb1), (cbt2, cb2)):
        cbt = cbt_ref[...]
        cn = jnp.sum(cbt * cbt, axis=0, keepdims=True)
        s = cn - 2.0 * _dot(r, cbt)
        m = jnp.min(s, axis=1, keepdims=True)
        diff = s - m
        e = jnp.exp2(diff * rt2)
        emb = _dot(e, cb_ref[...]) * (1.0 / jnp.sum(e, axis=1,
                                                    keepdims=True))
        lane = jax.lax.broadcasted_iota(jnp.int32, s.shape, 1)
        idv = jax.lax.bitcast_convert_type(diff, jnp.int32) | lane
        ids = jnp.min(idv, axis=1, keepdims=True)
        key = key * _K + ids
        diff = emb - r
        qloss = qloss + (1.0 + _COMMIT_W) * jnp.sum(diff * diff, axis=1,
                                                    keepdims=True)
        norms.append(jnp.sqrt(jnp.sum(emb * emb, axis=1, keepdims=True)))
        r = r - emb
        zhat = zhat + emb

    g = _silu(_dt(zhat, dw0[...]) + db0[...])
    g = _silu(_dt(g, dw1[...]) + db1[...])
    g = _silu(_dt(g, dw2[...]) + db2[...])
    xh = _dt(g, dw3[...]) + db3[...]

    nrm = jnp.sqrt(jnp.sum(xh * xh, axis=1, keepdims=True))
    xh = xh * (1.0 / (nrm + 1e-12))
    nc = _IN - _N_CAT
    xc = xh[:, :nc]
    cnrm = jnp.sqrt(jnp.sum(xc * xc, axis=1, keepdims=True))
    dm = xc * (1.0 / (cnrm + 1e-12)) - x[:, :nc]
    mse = jnp.sum(dm * dm, axis=1, keepdims=True)
    lg = xh[:, nc:]
    tg = x[:, nc:]
    bce_el = (jnp.maximum(lg, 0.0) - lg * tg
              + jnp.log(1.0 + jnp.exp(-jnp.abs(lg))))
    bce = jnp.sum(bce_el, axis=1, keepdims=True)
    recon = mse + bce

    @pl.when(i == 0)
    def _():
        rsum_ref[...] = jnp.zeros_like(rsum_ref)
        qsum_ref[...] = jnp.zeros_like(qsum_ref)

    rsum_ref[...] += jnp.sum(recon, axis=0, keepdims=True)
    qsum_ref[...] += jnp.sum(qloss, axis=0, keepdims=True)
    en_ref[...] = jnp.concatenate(norms, axis=1)
    key_ref[...] = key


def _uniq_body(keys_ref, keysT_ref, out_ref):
    i = pl.program_id(0)
    ki = keys_ref[...]
    kj = keysT_ref[...]
    mu = jnp.sum(jnp.where(ki == kj, 1.0, 0.0), axis=1, keepdims=True)
    cnt = jnp.sum(1.0 / mu, axis=0, keepdims=True)

    @pl.when(i == 0)
    def _():
        out_ref[...] = jnp.zeros_like(out_ref)

    out_ref[...] += cnt

    @pl.when(i == pl.num_programs(0) - 1)
    def _():
        out_ref[...] = jnp.floor(out_ref[...] + 0.5) / jnp.float32(
            kj.shape[1])


def kernel(x, gumbel_t,
           enc_W0, enc_b0, enc_W1, enc_b1, enc_W2, enc_b2, enc_W3, enc_b3,
           dec_W0, dec_b0, dec_W1, dec_b1, dec_W2, dec_b2, dec_W3, dec_b3,
           cb0, cb1, cb2):
    B = x.shape[0]
    row = lambda v: v.reshape(1, -1)
    bf = lambda v: v.astype(jnp.bfloat16)
    t = jnp.asarray(gumbel_t, jnp.float32).reshape(1, 1)

    nb = B // _BLK
    rep = lambda shape: pl.BlockSpec(shape, lambda i: (0, 0))
    blk = lambda shape: pl.BlockSpec(shape, lambda i: (i, 0))
    in_specs = [
        rep((1, 1)),
        blk((_BLK, _IN)),
        rep((1024, _IN)), rep((1, 1024)),
        rep((512, 1024)), rep((1, 512)),
        rep((256, 512)), rep((1, 256)),
        rep((_D, 256)), rep((1, _D)),
        rep((256, _D)), rep((1, 256)),
        rep((512, 256)), rep((1, 512)),
        rep((1024, 512)), rep((1, 1024)),
        rep((_IN, 1024)), rep((1, _IN)),
        rep((_D, _K)), rep((_K, _D)),
        rep((_D, _K)), rep((_K, _D)),
        rep((_D, _K)), rep((_K, _D)),
    ]
    out_specs = [
        rep((1, 1)),
        rep((1, 1)),
        blk((_BLK, 3)),
        blk((_BLK, 1)),
    ]
    out_shape = [
        jax.ShapeDtypeStruct((1, 1), jnp.float32),
        jax.ShapeDtypeStruct((1, 1), jnp.float32),
        jax.ShapeDtypeStruct((B, 3), jnp.float32),
        jax.ShapeDtypeStruct((B, 1), jnp.int32),
    ]
    rsum, qsum, en, keys = pl.pallas_call(
        _rq_body,
        grid=(nb,),
        in_specs=in_specs,
        out_specs=out_specs,
        out_shape=out_shape,
        compiler_params=pltpu.CompilerParams(
            dimension_semantics=("arbitrary",)),
    )(t, x,
      bf(enc_W0), row(enc_b0), bf(enc_W1), row(enc_b1),
      bf(enc_W2), row(enc_b2), bf(enc_W3), row(enc_b3),
      bf(dec_W0), row(dec_b0), bf(dec_W1), row(dec_b1),
      bf(dec_W2), row(dec_b2), bf(dec_W3), row(dec_b3),
      cb0.T, bf(cb0), cb1.T, bf(cb1), cb2.T, bf(cb2))

    keysT = keys.reshape(1, B)
    pu = pl.pallas_call(
        _uniq_body,
        grid=(B // _UBLK,),
        in_specs=[
            pl.BlockSpec((_UBLK, 1), lambda i: (i, 0)),
            pl.BlockSpec((1, B), lambda i: (0, 0)),
        ],
        out_specs=pl.BlockSpec((1, 1), lambda i: (0, 0)),
        out_shape=jax.ShapeDtypeStruct((1, 1), jnp.float32),
        compiler_params=pltpu.CompilerParams(
            dimension_semantics=("arbitrary",)),
    )(keys, keysT)

    recon_mean = rsum[0, 0] / B
    q_mean = qsum[0, 0] / B
    loss = recon_mean + q_mean
    return (loss, recon_mean, q_mean, en, pu[0, 0])


# speedup vs baseline: 3.6582x; 1.0357x over previous
import jax
import jax.numpy as jnp
from jax.experimental import pallas as pl
from jax.experimental.pallas import tpu as pltpu

_N_CAT = 18
_COMMIT_W = 0.25
_IN = 786
_D = 128
_K = 1024
_BLK = 1024
_UBLK = 512
_LOG2E = 1.4426950408889634


def _dot(a, b):
    return jax.lax.dot_general(a.astype(jnp.bfloat16), b.astype(jnp.bfloat16),
                               (((1,), (0,)), ((), ())),
                               preferred_element_type=jnp.float32)


def _dt(a, b):
    return jax.lax.dot_general(a.astype(jnp.bfloat16), b.astype(jnp.bfloat16),
                               (((1,), (1,)), ((), ())),
                               preferred_element_type=jnp.float32)


def _silu(h):
    u = 0.5 * h
    return u * (jnp.tanh(u) + 1.0)


def _rq_body(t_ref, x_ref,
             ew0, eb0, ew1, eb1, ew2, eb2, ew3, eb3,
             dw0, db0, dw1, db1, dw2, db2, dw3, db3,
             cbt0, cb0, cbt1, cb1, cbt2, cb2,
             rsum_ref, qsum_ref, en_ref, key_ref):
    i = pl.program_id(0)

    x = x_ref[...]
    h = _silu(_dt(x, ew0[...]) + eb0[...])
    h = _silu(_dt(h, ew1[...]) + eb1[...])
    h = _silu(_dt(h, ew2[...]) + eb2[...])
    z = _dt(h, ew3[...]) + eb3[...]

    rt2 = -_LOG2E / t_ref[...]
    r = z
    zhat = jnp.zeros_like(z)
    qloss = jnp.zeros((_BLK, 1), jnp.float32)
    key = jnp.zeros((_BLK, 1), jnp.int32)
    norms = []
    for cbt_ref, cb_ref in ((cbt0, cb0), (cbt1, cb1), (cbt2, cb2)):
        cbt = cbt_ref[...]
        cn = jnp.sum(cbt * cbt, axis=0, keepdims=True)
        s = cn - 2.0 * _dot(r, cbt)
        m = jnp.min(s, axis=1, keepdims=True)
        diff = s - m
        e = jnp.exp2(diff * rt2)
        emb = _dot(e, cb_ref[...]) * (1.0 / jnp.sum(e, axis=1,
                                                    keepdims=True))
        lane = jax.lax.broadcasted_iota(jnp.int32, s.shape, 1)
        idv = jax.lax.bitcast_convert_type(diff, jnp.int32) | lane
        ids = jnp.min(idv, axis=1, keepdims=True)
        key = key * _K + ids
        diff = emb - r
        qloss = qloss + (1.0 + _COMMIT_W) * jnp.sum(diff * diff, axis=1,
                                                    keepdims=True)
        norms.append(jnp.sqrt(jnp.sum(emb * emb, axis=1, keepdims=True)))
        r = r - emb
        zhat = zhat + emb

    g = _silu(_dt(zhat, dw0[...]) + db0[...])
    g = _silu(_dt(g, dw1[...]) + db1[...])
    g = _silu(_dt(g, dw2[...]) + db2[...])
    xh = _dt(g, dw3[...]) + db3[...]

    nrm = jnp.sqrt(jnp.sum(xh * xh, axis=1, keepdims=True))
    xh = xh * (1.0 / (nrm + 1e-12))
    nc = _IN - _N_CAT
    xc = xh[:, :nc]
    cnrm = jnp.sqrt(jnp.sum(xc * xc, axis=1, keepdims=True))
    dm = xc * (1.0 / (cnrm + 1e-12)) - x[:, :nc]
    mse = jnp.sum(dm * dm, axis=1, keepdims=True)
    lg = xh[:, nc:]
    tg = x[:, nc:]
    bce_el = (jnp.maximum(lg, 0.0) - lg * tg
              + jnp.log(1.0 + jnp.exp(-jnp.abs(lg))))
    bce = jnp.sum(bce_el, axis=1, keepdims=True)
    recon = mse + bce

    @pl.when(i == 0)
    def _():
        rsum_ref[...] = jnp.zeros_like(rsum_ref)
        qsum_ref[...] = jnp.zeros_like(qsum_ref)

    rsum_ref[...] += jnp.sum(recon, axis=0, keepdims=True)
    qsum_ref[...] += jnp.sum(qloss, axis=0, keepdims=True)
    en_ref[...] = jnp.concatenate(norms, axis=1)
    key_ref[...] = key


def _uniq_body(keys_ref, keysT_ref, out_ref):
    i = pl.program_id(0)
    ki = keys_ref[...]
    kj = keysT_ref[...]
    mu = jnp.sum(jnp.where(ki == kj, 1.0, 0.0), axis=1, keepdims=True)
    cnt = jnp.sum(1.0 / mu, axis=0, keepdims=True)

    @pl.when(i == 0)
    def _():
        out_ref[...] = jnp.zeros_like(out_ref)

    out_ref[...] += cnt

    @pl.when(i == pl.num_programs(0) - 1)
    def _():
        out_ref[...] = jnp.floor(out_ref[...] + 0.5) / jnp.float32(
            kj.shape[1])


def kernel(x, gumbel_t,
           enc_W0, enc_b0, enc_W1, enc_b1, enc_W2, enc_b2, enc_W3, enc_b3,
           dec_W0, dec_b0, dec_W1, dec_b1, dec_W2, dec_b2, dec_W3, dec_b3,
           cb0, cb1, cb2):
    B = x.shape[0]
    row = lambda v: v.reshape(1, -1)
    bf = lambda v: v.astype(jnp.bfloat16)
    t = jnp.asarray(gumbel_t, jnp.float32).reshape(1, 1)

    nb = B // _BLK
    rep = lambda shape: pl.BlockSpec(shape, lambda i: (0, 0))
    blk = lambda shape: pl.BlockSpec(shape, lambda i: (i, 0))
    in_specs = [
        rep((1, 1)),
        blk((_BLK, _IN)),
        rep((1024, _IN)), rep((1, 1024)),
        rep((512, 1024)), rep((1, 512)),
        rep((256, 512)), rep((1, 256)),
        rep((_D, 256)), rep((1, _D)),
        rep((256, _D)), rep((1, 256)),
        rep((512, 256)), rep((1, 512)),
        rep((1024, 512)), rep((1, 1024)),
        rep((_IN, 1024)), rep((1, _IN)),
        rep((_D, _K)), rep((_K, _D)),
        rep((_D, _K)), rep((_K, _D)),
        rep((_D, _K)), rep((_K, _D)),
    ]
    out_specs = [
        rep((1, 1)),
        rep((1, 1)),
        blk((_BLK, 3)),
        blk((_BLK, 1)),
    ]
    out_shape = [
        jax.ShapeDtypeStruct((1, 1), jnp.float32),
        jax.ShapeDtypeStruct((1, 1), jnp.float32),
        jax.ShapeDtypeStruct((B, 3), jnp.float32),
        jax.ShapeDtypeStruct((B, 1), jnp.int32),
    ]
    rsum, qsum, en, keys = pl.pallas_call(
        _rq_body,
        grid=(nb,),
        in_specs=in_specs,
        out_specs=out_specs,
        out_shape=out_shape,
        compiler_params=pltpu.CompilerParams(
            dimension_semantics=("arbitrary",)),
    )(t, x,
      bf(enc_W0), row(enc_b0), bf(enc_W1), row(enc_b1),
      bf(enc_W2), row(enc_b2), bf(enc_W3), row(enc_b3),
      bf(dec_W0), row(dec_b0), bf(dec_W1), row(dec_b1),
      bf(dec_W2), row(dec_b2), bf(dec_W3), row(dec_b3),
      cb0.T, bf(cb0), cb1.T, bf(cb1), cb2.T, bf(cb2))

    keysT = keys.reshape(1, B)
    pu = pl.pallas_call(
        _uniq_body,
        grid=(B // _UBLK,),
        in_specs=[
            pl.BlockSpec((_UBLK, 1), lambda i: (i, 0)),
            pl.BlockSpec((1, B), lambda i: (0, 0)),
        ],
        out_specs=pl.BlockSpec((1, 1), lambda i: (0, 0)),
        out_shape=jax.ShapeDtypeStruct((1, 1), jnp.float32),
        compiler_params=pltpu.CompilerParams(
            dimension_semantics=("arbitrary",)),
    )(keys, keysT)

    recon_mean = rsum[0, 0] / B
    q_mean = qsum[0, 0] / B
    loss = recon_mean + q_mean
    return (loss, recon_mean, q_mean, en, pu[0, 0])


# speedup vs baseline: 3.7544x; 1.0263x over previous
import jax
import jax.numpy as jnp
from jax.experimental import pallas as pl
from jax.experimental.pallas import tpu as pltpu

_N_CAT = 18
_COMMIT_W = 0.25
_IN = 786
_D = 128
_K = 1024
_BLK = 1024
_UBLK = 512
_LOG2E = 1.4426950408889634


def _dot(a, b):
    return jax.lax.dot_general(a.astype(jnp.bfloat16), b.astype(jnp.bfloat16),
                               (((1,), (0,)), ((), ())),
                               preferred_element_type=jnp.float32)


def _dt(a, b):
    return jax.lax.dot_general(a.astype(jnp.bfloat16), b.astype(jnp.bfloat16),
                               (((1,), (1,)), ((), ())),
                               preferred_element_type=jnp.float32)


def _silu(h):
    u = 0.5 * h
    return u * (jnp.tanh(u) + 1.0)


def _rq_body(q_ref, x_ref,
             ew0, eb0, ew1, eb1, ew2, eb2, ew3, eb3,
             dw0, db0, dw1, db1, dw2, db2, dw3, db3,
             cbt0, cb0, cbt1, cb1, cbt2, cb2,
             rsum_ref, qsum_ref, en_ref, key_ref, keyT_ref):
    i = pl.program_id(0)

    x = x_ref[...]
    h = _silu(_dt(x, ew0[...]) + eb0[...])
    h = _silu(_dt(h, ew1[...]) + eb1[...])
    h = _silu(_dt(h, ew2[...]) + eb2[...])
    z = _dt(h, ew3[...]) + eb3[...]

    q = q_ref[...]
    r = z
    zhat = jnp.zeros_like(z)
    qloss = jnp.zeros((_BLK, 1), jnp.float32)
    key = jnp.zeros((_BLK, 1), jnp.int32)
    rev = (_K - 1) - jax.lax.broadcasted_iota(jnp.int32, (_BLK, _K), 1)
    norms = []
    for cbw_ref, cb_ref in ((cbt0, cb0), (cbt1, cb1), (cbt2, cb2)):
        cbw = cbw_ref[...]
        cn2 = q * jnp.sum(cbw * cbw, axis=0, keepdims=True)
        s2 = cn2 + _dot(r, cbw)
        m2 = jnp.max(s2, axis=1, keepdims=True)
        u = s2 - m2
        e = jnp.exp2(u)
        emb = _dot(e, cb_ref[...]) * (1.0 / jnp.sum(e, axis=1,
                                                    keepdims=True))
        idv = jax.lax.bitcast_convert_type(u, jnp.int32) | rev
        ids = (_K - 1) - jnp.max(idv, axis=1, keepdims=True)
        key = key * _K + ids
        diff = emb - r
        qloss = qloss + (1.0 + _COMMIT_W) * jnp.sum(diff * diff, axis=1,
                                                    keepdims=True)
        norms.append(jnp.sqrt(jnp.sum(emb * emb, axis=1, keepdims=True)))
        r = r - emb
        zhat = zhat + emb

    g = _silu(_dt(zhat, dw0[...]) + db0[...])
    g = _silu(_dt(g, dw1[...]) + db1[...])
    g = _silu(_dt(g, dw2[...]) + db2[...])
    xh = _dt(g, dw3[...]) + db3[...]

    nrm = jnp.sqrt(jnp.sum(xh * xh, axis=1, keepdims=True))
    xh = xh * (1.0 / (nrm + 1e-12))
    nc = _IN - _N_CAT
    xc = xh[:, :nc]
    cnrm = jnp.sqrt(jnp.sum(xc * xc, axis=1, keepdims=True))
    dm = xc * (1.0 / (cnrm + 1e-12)) - x[:, :nc]
    mse = jnp.sum(dm * dm, axis=1, keepdims=True)
    lg = xh[:, nc:]
    tg = x[:, nc:]
    bce_el = (jnp.maximum(lg, 0.0) - lg * tg
              + jnp.log(1.0 + jnp.exp(-jnp.abs(lg))))
    bce = jnp.sum(bce_el, axis=1, keepdims=True)
    recon = mse + bce

    @pl.when(i == 0)
    def _():
        rsum_ref[...] = jnp.zeros_like(rsum_ref)
        qsum_ref[...] = jnp.zeros_like(qsum_ref)

    rsum_ref[...] += jnp.sum(recon, axis=0, keepdims=True)
    qsum_ref[...] += jnp.sum(qloss, axis=0, keepdims=True)
    en_ref[...] = jnp.concatenate(norms, axis=1)
    key_ref[...] = key
    keyT_ref[...] = jnp.swapaxes(key, 0, 1)


def _uniq_body(keys_ref, keysT_ref, out_ref):
    i = pl.program_id(0)
    ki = keys_ref[...]
    kj = keysT_ref[...]
    mu = jnp.sum(jnp.where(ki == kj, 1.0, 0.0), axis=1, keepdims=True)
    cnt = jnp.sum(1.0 / mu, axis=0, keepdims=True)

    @pl.when(i == 0)
    def _():
        out_ref[...] = jnp.zeros_like(out_ref)

    out_ref[...] += cnt

    @pl.when(i == pl.num_programs(0) - 1)
    def _():
        out_ref[...] = jnp.floor(out_ref[...] + 0.5) / jnp.float32(
            kj.shape[1])


def kernel(x, gumbel_t,
           enc_W0, enc_b0, enc_W1, enc_b1, enc_W2, enc_b2, enc_W3, enc_b3,
           dec_W0, dec_b0, dec_W1, dec_b1, dec_W2, dec_b2, dec_W3, dec_b3,
           cb0, cb1, cb2):
    B = x.shape[0]
    row = lambda v: v.reshape(1, -1)
    bf = lambda v: v.astype(jnp.bfloat16)
    t = jnp.asarray(gumbel_t, jnp.float32)
    a = (2.0 * _LOG2E) / t
    q = (-t / (4.0 * _LOG2E)).reshape(1, 1)

    nb = B // _BLK
    rep = lambda shape: pl.BlockSpec(shape, lambda i: (0, 0))
    blk = lambda shape: pl.BlockSpec(shape, lambda i: (i, 0))
    in_specs = [
        rep((1, 1)),
        blk((_BLK, _IN)),
        rep((1024, _IN)), rep((1, 1024)),
        rep((512, 1024)), rep((1, 512)),
        rep((256, 512)), rep((1, 256)),
        rep((_D, 256)), rep((1, _D)),
        rep((256, _D)), rep((1, 256)),
        rep((512, 256)), rep((1, 512)),
        rep((1024, 512)), rep((1, 1024)),
        rep((_IN, 1024)), rep((1, _IN)),
        rep((_D, _K)), rep((_K, _D)),
        rep((_D, _K)), rep((_K, _D)),
        rep((_D, _K)), rep((_K, _D)),
    ]
    out_specs = [
        rep((1, 1)),
        rep((1, 1)),
        blk((_BLK, 3)),
        blk((_BLK, 1)),
        pl.BlockSpec((1, _BLK), lambda i: (0, i)),
    ]
    out_shape = [
        jax.ShapeDtypeStruct((1, 1), jnp.float32),
        jax.ShapeDtypeStruct((1, 1), jnp.float32),
        jax.ShapeDtypeStruct((B, 3), jnp.float32),
        jax.ShapeDtypeStruct((B, 1), jnp.int32),
        jax.ShapeDtypeStruct((1, B), jnp.int32),
    ]
    rsum, qsum, en, keys, keysT = pl.pallas_call(
        _rq_body,
        grid=(nb,),
        in_specs=in_specs,
        out_specs=out_specs,
        out_shape=out_shape,
        compiler_params=pltpu.CompilerParams(
            dimension_semantics=("arbitrary",)),
    )(q, x,
      bf(enc_W0), row(enc_b0), bf(enc_W1), row(enc_b1),
      bf(enc_W2), row(enc_b2), bf(enc_W3), row(enc_b3),
      bf(dec_W0), row(dec_b0), bf(dec_W1), row(dec_b1),
      bf(dec_W2), row(dec_b2), bf(dec_W3), row(dec_b3),
      a * cb0.T, bf(cb0), a * cb1.T, bf(cb1), a * cb2.T, bf(cb2))

    pu = pl.pallas_call(
        _uniq_body,
        grid=(B // _UBLK,),
        in_specs=[
            pl.BlockSpec((_UBLK, 1), lambda i: (i, 0)),
            pl.BlockSpec((1, B), lambda i: (0, 0)),
        ],
        out_specs=pl.BlockSpec((1, 1), lambda i: (0, 0)),
        out_shape=jax.ShapeDtypeStruct((1, 1), jnp.float32),
        compiler_params=pltpu.CompilerParams(
            dimension_semantics=("arbitrary",)),
    )(keys, keysT)

    recon_mean = rsum[0, 0] / B
    q_mean = qsum[0, 0] / B
    loss = recon_mean + q_mean
    return (loss, recon_mean, q_mean, en, pu[0, 0])
